# baseline (speedup 1.0000x reference)
import jax
import jax.numpy as jnp
from jax import lax
from jax.experimental import pallas as pl
from jax.experimental.pallas import tpu as pltpu
from jax.experimental.pallas import tpu_sc as plsc

NC = 2
NS = 16
L = 16
NW = NC * NS
NODES = 2048
PAIRS = NODES // 2
TOK = 2 * 8192
ROWS_W = TOK // NW
T = 8
GROUPS = ROWS_W // T
CHUNKS = PAIRS // L
TN = T * NODES
NB = 2


def _rot_body(inp_hbm, ii_hbm, jj_hbm, oi_hbm, oj_hbm, c_hbm, s_hbm,
              out_hbm, ii_v, jj_v, oi_v, oj_v, c_v, s_v,
              in0, in1, out0, out1, isem0, isem1, osem0, osem1):
    in_bufs = (in0, in1)
    out_bufs = (out0, out1)
    isems = (isem0, isem1)
    osems = (osem0, osem1)

    wid = lax.axis_index("s") * NC + lax.axis_index("c")
    base_e = wid * ROWS_W * NODES

    pltpu.sync_copy(ii_hbm, ii_v)
    pltpu.sync_copy(jj_hbm, jj_v)
    pltpu.sync_copy(oi_hbm, oi_v)
    pltpu.sync_copy(oj_hbm, oj_v)
    pltpu.sync_copy(c_hbm, c_v)
    pltpu.sync_copy(s_hbm, s_v)

    pltpu.async_copy(inp_hbm.at[pl.ds(base_e, TN)], in_bufs[0], isems[0])

    def step(h, carry):
        for b in range(NB):
            g = h * NB + b
            e0 = base_e + g * TN
            nb = (b + 1) % NB

            @pl.when(g + 1 < GROUPS)
            def _prefetch():
                pltpu.async_copy(inp_hbm.at[pl.ds(e0 + TN, TN)],
                                 in_bufs[nb], isems[nb])

            pltpu.make_async_copy(inp_hbm.at[pl.ds(e0, TN)],
                                  in_bufs[b], isems[b]).wait()

            @pl.when(h > 0)
            def _drain_out():
                pltpu.make_async_copy(
                    out_bufs[b], out_hbm.at[pl.ds(e0 - NB * TN, TN)],
                    osems[b]).wait()

            @plsc.parallel_loop(0, PAIRS, step=L, unroll=8)
            def chunk(o):
                ii = ii_v[pl.ds(o, L)]
                jj = jj_v[pl.ds(o, L)]
                oi = oi_v[pl.ds(o, L)]
                oj = oj_v[pl.ds(o, L)]
                cc = c_v[pl.ds(o, L)]
                ss = s_v[pl.ds(o, L)]
                for tt in range(T):
                    src = in_bufs[b].at[pl.ds(tt * NODES, NODES)]
                    dst = out_bufs[b].at[pl.ds(tt * NODES, NODES)]
                    xi = plsc.load_gather(src, [ii])
                    xj = plsc.load_gather(src, [jj])
                    yi = cc * xi - ss * xj
                    yj = cc * xj + ss * xi
                    plsc.store_scatter(dst, [oi], yi)
                    plsc.store_scatter(dst, [oj], yj)
            pltpu.async_copy(out_bufs[b], out_hbm.at[pl.ds(e0, TN)],
                             osems[b])
        return carry

    lax.fori_loop(0, GROUPS // NB, step, 0)

    for b in range(NB):
        e0 = base_e + (GROUPS - NB + b) * TN
        pltpu.make_async_copy(out_bufs[b], out_hbm.at[pl.ds(e0, TN)],
                              osems[b]).wait()


def kernel(inp, angles, pairs, outp_pairs):
    c = jnp.cos(angles)
    s = jnp.sin(angles)
    ii = pairs[:, 0]
    jj = pairs[:, 1]
    oi = outp_pairs[:, 0]
    oj = outp_pairs[:, 1]
    flat = inp.reshape(TOK * NODES)

    run = pl.kernel(
        _rot_body,
        out_type=jax.ShapeDtypeStruct((TOK * NODES,), jnp.float32),
        mesh=plsc.VectorSubcoreMesh(
            core_axis_name="c", subcore_axis_name="s",
            num_cores=NC, num_subcores=NS),
        compiler_params=pltpu.CompilerParams(needs_layout_passes=False),
        scratch_types=[
            pltpu.VMEM((PAIRS,), jnp.int32),
            pltpu.VMEM((PAIRS,), jnp.int32),
            pltpu.VMEM((PAIRS,), jnp.int32),
            pltpu.VMEM((PAIRS,), jnp.int32),
            pltpu.VMEM((PAIRS,), jnp.float32),
            pltpu.VMEM((PAIRS,), jnp.float32),
            pltpu.VMEM((TN,), jnp.float32),
            pltpu.VMEM((TN,), jnp.float32),
            pltpu.VMEM((TN,), jnp.float32),
            pltpu.VMEM((TN,), jnp.float32),
            pltpu.SemaphoreType.DMA,
            pltpu.SemaphoreType.DMA,
            pltpu.SemaphoreType.DMA,
            pltpu.SemaphoreType.DMA,
        ],
    )
    out = run(flat, ii, jj, oi, oj, c, s)
    return out.reshape(inp.shape)


# speedup vs baseline: 1.5181x; 1.5181x over previous
import jax
import jax.numpy as jnp
from jax import lax
from jax.experimental import pallas as pl
from jax.experimental.pallas import tpu as pltpu
from jax.experimental.pallas import tpu_sc as plsc

NC = 2
NS = 16
L = 16
NW = NC * NS
NODES = 2048
PAIRS = NODES // 2
TOK = 2 * 8192
ROWS_W = TOK // NW
T = 8
GROUPS = ROWS_W // T
CHUNKS = PAIRS // L
TN = T * NODES
NB = 2


def _rot_body(inp_hbm, ii_hbm, jj_hbm, oi_hbm, oj_hbm, c_hbm, s_hbm,
              out_hbm, ii_v, jj_v, oi_v, oj_v, c_v, s_v,
              in0, in1, out0, out1, isem0, isem1, osem0, osem1):
    in_bufs = (in0, in1)
    out_bufs = (out0, out1)
    isems = (isem0, isem1)
    osems = (osem0, osem1)

    wid = lax.axis_index("s") * NC + lax.axis_index("c")
    base_e = wid * ROWS_W * NODES

    pltpu.sync_copy(ii_hbm, ii_v)
    pltpu.sync_copy(jj_hbm, jj_v)
    pltpu.sync_copy(oi_hbm, oi_v)
    pltpu.sync_copy(oj_hbm, oj_v)
    pltpu.sync_copy(c_hbm, c_v)
    pltpu.sync_copy(s_hbm, s_v)

    pltpu.async_copy(inp_hbm.at[pl.ds(base_e, TN)], in_bufs[0], isems[0])

    def step(h, carry):
        for b in range(NB):
            g = h * NB + b
            e0 = base_e + g * TN
            nb = (b + 1) % NB

            @pl.when(g + 1 < GROUPS)
            def _prefetch():
                pltpu.async_copy(inp_hbm.at[pl.ds(e0 + TN, TN)],
                                 in_bufs[nb], isems[nb])

            pltpu.make_async_copy(inp_hbm.at[pl.ds(e0, TN)],
                                  in_bufs[b], isems[b]).wait()

            @pl.when(h > 0)
            def _drain_out():
                pltpu.make_async_copy(
                    out_bufs[b], out_hbm.at[pl.ds(e0 - NB * TN, TN)],
                    osems[b]).wait()

            @plsc.parallel_loop(0, PAIRS, step=L, unroll=4)
            def chunk(o):
                ii = ii_v[pl.ds(o, L)]
                jj = jj_v[pl.ds(o, L)]
                oi = oi_v[pl.ds(o, L)]
                oj = oj_v[pl.ds(o, L)]
                cc = c_v[pl.ds(o, L)]
                ss = s_v[pl.ds(o, L)]
                for tt in range(T):
                    src = in_bufs[b].at[pl.ds(tt * NODES, NODES)]
                    dst = out_bufs[b].at[pl.ds(tt * NODES, NODES)]
                    xi = plsc.load_gather(src, [ii])
                    xj = plsc.load_gather(src, [jj])
                    yi = cc * xi - ss * xj
                    yj = cc * xj + ss * xi
                    plsc.store_scatter(dst, [oi], yi)
                    plsc.store_scatter(dst, [oj], yj)
            pltpu.async_copy(out_bufs[b], out_hbm.at[pl.ds(e0, TN)],
                             osems[b])
        return carry

    lax.fori_loop(0, GROUPS // NB, step, 0)

    for b in range(NB):
        e0 = base_e + (GROUPS - NB + b) * TN
        pltpu.make_async_copy(out_bufs[b], out_hbm.at[pl.ds(e0, TN)],
                              osems[b]).wait()


def kernel(inp, angles, pairs, outp_pairs):
    c = jnp.cos(angles)
    s = jnp.sin(angles)
    ii = pairs[:, 0]
    jj = pairs[:, 1]
    oi = outp_pairs[:, 0]
    oj = outp_pairs[:, 1]
    flat = inp.reshape(TOK * NODES)

    run = pl.kernel(
        _rot_body,
        out_type=jax.ShapeDtypeStruct((TOK * NODES,), jnp.float32),
        mesh=plsc.VectorSubcoreMesh(
            core_axis_name="c", subcore_axis_name="s",
            num_cores=NC, num_subcores=NS),
        compiler_params=pltpu.CompilerParams(needs_layout_passes=False),
        scratch_types=[
            pltpu.VMEM((PAIRS,), jnp.int32),
            pltpu.VMEM((PAIRS,), jnp.int32),
            pltpu.VMEM((PAIRS,), jnp.int32),
            pltpu.VMEM((PAIRS,), jnp.int32),
            pltpu.VMEM((PAIRS,), jnp.float32),
            pltpu.VMEM((PAIRS,), jnp.float32),
            pltpu.VMEM((TN,), jnp.float32),
            pltpu.VMEM((TN,), jnp.float32),
            pltpu.VMEM((TN,), jnp.float32),
            pltpu.VMEM((TN,), jnp.float32),
            pltpu.SemaphoreType.DMA,
            pltpu.SemaphoreType.DMA,
            pltpu.SemaphoreType.DMA,
            pltpu.SemaphoreType.DMA,
        ],
    )
    out = run(flat, ii, jj, oi, oj, c, s)
    return out.reshape(inp.shape)
